# baseline (speedup 1.0000x reference)
import functools

import jax
import jax.numpy as jnp
from jax import lax
from jax.experimental import pallas as pl
from jax.experimental.pallas import tpu as pltpu
from jax.experimental.pallas import tpu_sc as plsc


def _sc_gather(table, idx):
  V, D = table.shape
  B = idx.shape[0]
  info = plsc.get_sparse_core_info()
  NC, NS = info.num_cores, info.num_subcores
  NW = NC * NS
  assert B % NW == 0 and (B // NW) % 8 == 0
  b_per_w = B // NW

  mesh = plsc.VectorSubcoreMesh(core_axis_name="c", subcore_axis_name="s")

  @functools.partial(
      pl.kernel,
      mesh=mesh,
      out_type=jax.ShapeDtypeStruct((B, D), jnp.float32),
      scratch_types=[
          pltpu.VMEM((b_per_w,), jnp.int32),
          pltpu.VMEM((b_per_w, D), jnp.float32),
          pltpu.SemaphoreType.DMA,
      ],
      compiler_params=pltpu.CompilerParams(use_tc_tiling_on_sc=False),
  )
  def gather_kernel(table_hbm, idx_hbm, out_hbm, idx_v, rows_v, sem):
    wid = lax.axis_index("s") * NC + lax.axis_index("c")
    base = wid * b_per_w
    pltpu.sync_copy(idx_hbm.at[pl.ds(base, b_per_w)], idx_v)
    pltpu.async_copy(table_hbm.at[idx_v], rows_v, sem).wait()
    pltpu.sync_copy(rows_v, out_hbm.at[pl.ds(base, b_per_w)])

  return gather_kernel(table, idx)


_V_TILE = 2048
_NBUF = 4


_B_TILE = 16
_NBUF = 4


def _make_proj(B, E, V):
  ngrid = B // _B_TILE

  def body(emb_ref, wt_ref, b_ref, out_hbm, *scratch):
    bufs = scratch[:_NBUF]
    sems = scratch[_NBUF:]
    i = pl.program_id(0)
    phase = jax.lax.rem(i, _NBUF)
    acc = jax.lax.dot_general(
        emb_ref[...],
        wt_ref[...],
        dimension_numbers=(((1,), (0,)), ((), ())),
        preferred_element_type=jnp.float32,
    ) + b_ref[...]

    for k in range(_NBUF):
      @pl.when(phase == k)
      def _(k=k):
        @pl.when(i >= _NBUF)
        def _():
          pltpu.make_async_copy(
              bufs[k], out_hbm.at[pl.ds((i - _NBUF) * _B_TILE, _B_TILE), :],
              sems[k]).wait()
        bufs[k][...] = acc
        pltpu.make_async_copy(
            bufs[k], out_hbm.at[pl.ds(i * _B_TILE, _B_TILE), :],
            sems[k]).start()

    @pl.when(i == ngrid - 1)
    def _():
      for k in range(_NBUF):
        pltpu.make_async_copy(
            bufs[k], out_hbm.at[pl.ds(0, _B_TILE), :], sems[k]).wait()

  return pl.pallas_call(
      body,
      grid=(ngrid,),
      in_specs=[
          pl.BlockSpec((_B_TILE, E), lambda i: (i, 0)),
          pl.BlockSpec((E, V), lambda i: (0, 0)),
          pl.BlockSpec((1, V), lambda i: (0, 0)),
      ],
      out_specs=pl.BlockSpec(memory_space=pl.ANY),
      out_shape=jax.ShapeDtypeStruct((B, V), jnp.float32),
      scratch_shapes=(
          [pltpu.VMEM((_B_TILE, V), jnp.float32) for _ in range(_NBUF)]
          + [pltpu.SemaphoreType.DMA for _ in range(_NBUF)]
      ),
  )


_T_TILE = 8192


def _transpose_body(w_ref, wt_ref):
  wt_ref[...] = jnp.swapaxes(w_ref[...], 0, 1)


def _tc_transpose(W):
  V, E = W.shape
  nt = pl.cdiv(V, _T_TILE)
  return pl.pallas_call(
      _transpose_body,
      grid=(nt,),
      in_specs=[pl.BlockSpec((_T_TILE, E), lambda i: (i, 0))],
      out_specs=pl.BlockSpec((E, _T_TILE), lambda i: (0, i)),
      out_shape=jax.ShapeDtypeStruct((E, V), jnp.float32),
  )(W)


def _tc_project(emb, W, b2d):
  B, E = emb.shape
  V = W.shape[0]
  return _make_proj(B, E, V)(emb, _tc_transpose(W), b2d)


def kernel(center_word, emb_table, W, b):
  idx = center_word.astype(jnp.int32)
  emb = jnp.take(emb_table, idx, axis=0)
  return _tc_project(emb, W, b.reshape(1, -1))


# speedup vs baseline: 1.0010x; 1.0010x over previous
import functools

import jax
import jax.numpy as jnp
from jax import lax
from jax.experimental import pallas as pl
from jax.experimental.pallas import tpu as pltpu
from jax.experimental.pallas import tpu_sc as plsc


def _sc_gather(table, idx):
  V, D = table.shape
  B = idx.shape[0]
  info = plsc.get_sparse_core_info()
  NC, NS = info.num_cores, info.num_subcores
  NW = NC * NS
  assert B % NW == 0 and (B // NW) % 8 == 0
  b_per_w = B // NW

  mesh = plsc.VectorSubcoreMesh(core_axis_name="c", subcore_axis_name="s")

  @functools.partial(
      pl.kernel,
      mesh=mesh,
      out_type=jax.ShapeDtypeStruct((B, D), jnp.float32),
      scratch_types=[
          pltpu.VMEM((b_per_w,), jnp.int32),
          pltpu.VMEM((b_per_w, D), jnp.float32),
          pltpu.SemaphoreType.DMA,
      ],
      compiler_params=pltpu.CompilerParams(use_tc_tiling_on_sc=False),
  )
  def gather_kernel(table_hbm, idx_hbm, out_hbm, idx_v, rows_v, sem):
    wid = lax.axis_index("s") * NC + lax.axis_index("c")
    base = wid * b_per_w
    pltpu.sync_copy(idx_hbm.at[pl.ds(base, b_per_w)], idx_v)
    pltpu.async_copy(table_hbm.at[idx_v], rows_v, sem).wait()
    pltpu.sync_copy(rows_v, out_hbm.at[pl.ds(base, b_per_w)])

  return gather_kernel(table, idx)


_V_TILE = 2048
_NBUF = 4


_B_TILE = 16
_NBUF = 4


def _make_proj(B, E, V):
  ngrid = B // _B_TILE

  def body(emb_ref, wt_ref, b_ref, out_hbm, *scratch):
    bufs = scratch[:_NBUF]
    sems = scratch[_NBUF:]
    i = pl.program_id(0)
    phase = jax.lax.rem(i, _NBUF)
    acc = jax.lax.dot_general(
        emb_ref[...],
        wt_ref[...],
        dimension_numbers=(((1,), (0,)), ((), ())),
        preferred_element_type=jnp.float32,
    ) + b_ref[...]

    for k in range(_NBUF):
      @pl.when(phase == k)
      def _(k=k):
        @pl.when(i >= _NBUF)
        def _():
          pltpu.make_async_copy(
              bufs[k], out_hbm.at[pl.ds((i - _NBUF) * _B_TILE, _B_TILE), :],
              sems[k]).wait()
        bufs[k][...] = acc
        pltpu.make_async_copy(
            bufs[k], out_hbm.at[pl.ds(i * _B_TILE, _B_TILE), :],
            sems[k]).start()

    @pl.when(i == ngrid - 1)
    def _():
      for k in range(_NBUF):
        pltpu.make_async_copy(
            bufs[k], out_hbm.at[pl.ds(0, _B_TILE), :], sems[k]).wait()

  return pl.pallas_call(
      body,
      grid=(ngrid,),
      in_specs=[
          pl.BlockSpec((_B_TILE, E), lambda i: (i, 0)),
          pl.BlockSpec((E, V), lambda i: (0, 0)),
          pl.BlockSpec((1, V), lambda i: (0, 0)),
      ],
      out_specs=pl.BlockSpec(memory_space=pl.ANY),
      out_shape=jax.ShapeDtypeStruct((B, V), jnp.float32),
      scratch_shapes=(
          [pltpu.VMEM((_B_TILE, V), jnp.float32) for _ in range(_NBUF)]
          + [pltpu.SemaphoreType.DMA for _ in range(_NBUF)]
      ),
      name="band_proj",
  )


_T_TILE = 8192


def _transpose_body(w_ref, wt_ref):
  wt_ref[...] = jnp.swapaxes(w_ref[...], 0, 1)


def _tc_transpose(W):
  V, E = W.shape
  nt = pl.cdiv(V, _T_TILE)
  return pl.pallas_call(
      _transpose_body,
      grid=(nt,),
      in_specs=[pl.BlockSpec((_T_TILE, E), lambda i: (i, 0))],
      out_specs=pl.BlockSpec((E, _T_TILE), lambda i: (0, i)),
      out_shape=jax.ShapeDtypeStruct((E, V), jnp.float32),
      name="wt_transpose",
  )(W)


def _tc_project(emb, W, b2d):
  B, E = emb.shape
  V = W.shape[0]
  return _make_proj(B, E, V)(emb, _tc_transpose(W), b2d)


def kernel(center_word, emb_table, W, b):
  idx = center_word.astype(jnp.int32)
  emb = jnp.take(emb_table, idx, axis=0)
  return _tc_project(emb, W, b.reshape(1, -1))


# speedup vs baseline: 2.7650x; 2.7621x over previous
import functools

import jax
import jax.numpy as jnp
from jax import lax
from jax.experimental import pallas as pl
from jax.experimental.pallas import tpu as pltpu
from jax.experimental.pallas import tpu_sc as plsc


def _sc_gather(table, idx):
  V, D = table.shape
  B = idx.shape[0]
  info = plsc.get_sparse_core_info()
  NC, NS = info.num_cores, info.num_subcores
  NW = NC * NS
  assert B % NW == 0 and (B // NW) % 8 == 0
  b_per_w = B // NW

  mesh = plsc.VectorSubcoreMesh(core_axis_name="c", subcore_axis_name="s")

  @functools.partial(
      pl.kernel,
      mesh=mesh,
      out_type=jax.ShapeDtypeStruct((B, D), jnp.float32),
      scratch_types=[
          pltpu.VMEM((b_per_w,), jnp.int32),
          pltpu.VMEM((b_per_w, D), jnp.float32),
          pltpu.SemaphoreType.DMA,
      ],
      compiler_params=pltpu.CompilerParams(use_tc_tiling_on_sc=False),
  )
  def gather_kernel(table_hbm, idx_hbm, out_hbm, idx_v, rows_v, sem):
    wid = lax.axis_index("s") * NC + lax.axis_index("c")
    base = wid * b_per_w
    pltpu.sync_copy(idx_hbm.at[pl.ds(base, b_per_w)], idx_v)
    pltpu.async_copy(table_hbm.at[idx_v], rows_v, sem).wait()
    pltpu.sync_copy(rows_v, out_hbm.at[pl.ds(base, b_per_w)])

  return gather_kernel(table, idx)


_V_TILE = 2048
_NBUF = 4


def _make_proj_t(B, E1, V):
  ngrid = pl.cdiv(V, _V_TILE)
  tail = V - (ngrid - 1) * _V_TILE

  def body(wb_ref, emb_ref, out_hbm, *scratch):
    bufs = scratch[:_NBUF]
    sems = scratch[_NBUF:]
    i = pl.program_id(0)
    phase = jax.lax.rem(i, _NBUF)
    acc = jax.lax.dot_general(
        wb_ref[...],
        emb_ref[...],
        dimension_numbers=(((0,), (1,)), ((), ())),
        preferred_element_type=jnp.float32,
    )

    for k in range(_NBUF):
      @pl.when(phase == k)
      def _(k=k):
        @pl.when(i >= _NBUF)
        def _():
          pltpu.make_async_copy(
              bufs[k], out_hbm.at[pl.ds((i - _NBUF) * _V_TILE, _V_TILE), :],
              sems[k]).wait()
        bufs[k][...] = acc
        @pl.when(i < ngrid - 1)
        def _():
          pltpu.make_async_copy(
              bufs[k], out_hbm.at[pl.ds(i * _V_TILE, _V_TILE), :],
              sems[k]).start()
        @pl.when(i == ngrid - 1)
        def _():
          pltpu.make_async_copy(
              bufs[k].at[pl.ds(0, tail), :],
              out_hbm.at[pl.ds(i * _V_TILE, tail), :],
              sems[k]).start()
          pltpu.make_async_copy(
              bufs[k].at[pl.ds(0, tail), :],
              out_hbm.at[pl.ds(i * _V_TILE, tail), :],
              sems[k]).wait()

    @pl.when(i == ngrid - 1)
    def _():
      for k in range(_NBUF):
        @pl.when(phase != k)
        def _(k=k):
          pltpu.make_async_copy(
              bufs[k], out_hbm.at[pl.ds(0, _V_TILE), :], sems[k]).wait()

  return pl.pallas_call(
      body,
      grid=(ngrid,),
      in_specs=[
          pl.BlockSpec((E1, _V_TILE), lambda i: (0, i)),
          pl.BlockSpec((B, E1), lambda i: (0, 0)),
      ],
      out_specs=pl.BlockSpec(memory_space=pl.ANY),
      out_shape=jax.ShapeDtypeStruct((V, B), jnp.float32),
      scratch_shapes=(
          [pltpu.VMEM((_V_TILE, B), jnp.float32) for _ in range(_NBUF)]
          + [pltpu.SemaphoreType.DMA for _ in range(_NBUF)]
      ),
      name="band_proj_t",
  )


def kernel(center_word, emb_table, W, b):
  idx = center_word.astype(jnp.int32)
  emb = _sc_gather(emb_table, idx)
  B, E = emb.shape
  V = W.shape[0]
  wt = jnp.swapaxes(W, 0, 1)
  wb = jnp.concatenate([wt, b[None, :]], axis=0)
  emb1 = jnp.concatenate([emb, jnp.ones((B, 1), jnp.float32)], axis=1)
  out_t = _make_proj_t(B, E + 1, V)(wb, emb1)
  return jnp.swapaxes(out_t, 0, 1)


# speedup vs baseline: 2.8017x; 1.0133x over previous
import functools

import jax
import jax.numpy as jnp
from jax import lax
from jax.experimental import pallas as pl
from jax.experimental.pallas import tpu as pltpu
from jax.experimental.pallas import tpu_sc as plsc


def _sc_gather(table, idx):
  V, D = table.shape
  B = idx.shape[0]
  info = plsc.get_sparse_core_info()
  NC, NS = info.num_cores, info.num_subcores
  NW = NC * NS
  assert B % NW == 0 and (B // NW) % 8 == 0
  b_per_w = B // NW

  mesh = plsc.VectorSubcoreMesh(core_axis_name="c", subcore_axis_name="s")

  @functools.partial(
      pl.kernel,
      mesh=mesh,
      out_type=jax.ShapeDtypeStruct((B, D), jnp.float32),
      scratch_types=[
          pltpu.VMEM((b_per_w,), jnp.int32),
          pltpu.VMEM((b_per_w, D), jnp.float32),
          pltpu.SemaphoreType.DMA,
      ],
      compiler_params=pltpu.CompilerParams(use_tc_tiling_on_sc=True),
  )
  def gather_kernel(table_hbm, idx_hbm, out_hbm, idx_v, rows_v, sem):
    wid = lax.axis_index("s") * NC + lax.axis_index("c")
    base = wid * b_per_w
    pltpu.sync_copy(idx_hbm.at[pl.ds(base, b_per_w)], idx_v)
    pltpu.async_copy(table_hbm.at[idx_v], rows_v, sem).wait()
    pltpu.sync_copy(rows_v, out_hbm.at[pl.ds(base, b_per_w)])

  return gather_kernel(table, idx)


_V_TILE = 2048
_NBUF = 4


def _make_proj_t(B, E1, V):
  ngrid = pl.cdiv(V, _V_TILE)
  tail = V - (ngrid - 1) * _V_TILE

  def body(wb_ref, emb_ref, out_hbm, *scratch):
    bufs = scratch[:_NBUF]
    sems = scratch[_NBUF:]
    i = pl.program_id(0)
    phase = jax.lax.rem(i, _NBUF)
    acc = jax.lax.dot_general(
        wb_ref[...],
        emb_ref[...],
        dimension_numbers=(((0,), (1,)), ((), ())),
        preferred_element_type=jnp.float32,
    )

    for k in range(_NBUF):
      @pl.when(phase == k)
      def _(k=k):
        @pl.when(i >= _NBUF)
        def _():
          pltpu.make_async_copy(
              bufs[k], out_hbm.at[pl.ds((i - _NBUF) * _V_TILE, _V_TILE), :],
              sems[k]).wait()
        bufs[k][...] = acc
        @pl.when(i < ngrid - 1)
        def _():
          pltpu.make_async_copy(
              bufs[k], out_hbm.at[pl.ds(i * _V_TILE, _V_TILE), :],
              sems[k]).start()
        @pl.when(i == ngrid - 1)
        def _():
          pltpu.make_async_copy(
              bufs[k].at[pl.ds(0, tail), :],
              out_hbm.at[pl.ds(i * _V_TILE, tail), :],
              sems[k]).start()
          pltpu.make_async_copy(
              bufs[k].at[pl.ds(0, tail), :],
              out_hbm.at[pl.ds(i * _V_TILE, tail), :],
              sems[k]).wait()

    @pl.when(i == ngrid - 1)
    def _():
      for k in range(_NBUF):
        @pl.when(phase != k)
        def _(k=k):
          pltpu.make_async_copy(
              bufs[k], out_hbm.at[pl.ds(0, _V_TILE), :], sems[k]).wait()

  return pl.pallas_call(
      body,
      grid=(ngrid,),
      in_specs=[
          pl.BlockSpec((E1, _V_TILE), lambda i: (0, i)),
          pl.BlockSpec((B, E1), lambda i: (0, 0)),
      ],
      out_specs=pl.BlockSpec(memory_space=pl.ANY),
      out_shape=jax.ShapeDtypeStruct((V, B), jnp.float32),
      scratch_shapes=(
          [pltpu.VMEM((_V_TILE, B), jnp.float32) for _ in range(_NBUF)]
          + [pltpu.SemaphoreType.DMA for _ in range(_NBUF)]
      ),
      name="band_proj_t",
  )


def kernel(center_word, emb_table, W, b):
  idx = center_word.astype(jnp.int32)
  V, E = emb_table.shape
  table_pad = jnp.pad(emb_table, ((0, 0), (0, 128 - E)))
  emb = _sc_gather(table_pad, idx)[:, :E]
  B = emb.shape[0]
  wt = jnp.swapaxes(W, 0, 1)
  wb = jnp.concatenate([wt, b[None, :]], axis=0)
  emb1 = jnp.concatenate([emb, jnp.ones((B, 1), jnp.float32)], axis=1)
  out_t = _make_proj_t(B, E + 1, V)(wb, emb1)
  return jnp.swapaxes(out_t, 0, 1)


# speedup vs baseline: 3.0543x; 1.0902x over previous
import functools

import jax
import jax.numpy as jnp
from jax import lax
from jax.experimental import pallas as pl
from jax.experimental.pallas import tpu as pltpu
from jax.experimental.pallas import tpu_sc as plsc

_LANES = 128


_TP_TILE = 4096
_TP_NBUF = 4


def _make_tpad(E, V):
  ngrid = pl.cdiv(V, _TP_TILE)
  tail = V - (ngrid - 1) * _TP_TILE

  def body(tt_ref, out_hbm, *scratch):
    bufs = scratch[:_TP_NBUF]
    sems = scratch[_TP_NBUF:]
    i = pl.program_id(0)
    phase = jax.lax.rem(i, _TP_NBUF)
    t = jnp.swapaxes(tt_ref[...], 0, 1)

    for k in range(_TP_NBUF):
      @pl.when(phase == k)
      def _(k=k):
        @pl.when(i >= _TP_NBUF)
        def _():
          pltpu.make_async_copy(
              bufs[k], out_hbm.at[pl.ds((i - _TP_NBUF) * _TP_TILE, _TP_TILE), :],
              sems[k]).wait()
        bufs[k][:, :E] = t
        bufs[k][:, E:E + 1] = jnp.ones((_TP_TILE, 1), jnp.float32)
        @pl.when(i < ngrid - 1)
        def _():
          pltpu.make_async_copy(
              bufs[k], out_hbm.at[pl.ds(i * _TP_TILE, _TP_TILE), :],
              sems[k]).start()
        @pl.when(i == ngrid - 1)
        def _():
          pltpu.make_async_copy(
              bufs[k].at[pl.ds(0, tail), :],
              out_hbm.at[pl.ds(i * _TP_TILE, tail), :],
              sems[k]).start()
          pltpu.make_async_copy(
              bufs[k].at[pl.ds(0, tail), :],
              out_hbm.at[pl.ds(i * _TP_TILE, tail), :],
              sems[k]).wait()

    @pl.when(i == ngrid - 1)
    def _():
      for k in range(_TP_NBUF):
        @pl.when(phase != k)
        def _(k=k):
          pltpu.make_async_copy(
              bufs[k], out_hbm.at[pl.ds(0, _TP_TILE), :], sems[k]).wait()

  return pl.pallas_call(
      body,
      grid=(ngrid,),
      in_specs=[pl.BlockSpec((E, _TP_TILE), lambda i: (0, i))],
      out_specs=pl.BlockSpec(memory_space=pl.ANY),
      out_shape=jax.ShapeDtypeStruct((V, _LANES), jnp.float32),
      scratch_shapes=(
          [pltpu.VMEM((_TP_TILE, _LANES), jnp.float32) for _ in range(_TP_NBUF)]
          + [pltpu.SemaphoreType.DMA for _ in range(_TP_NBUF)]
      ),
      name="tpad",
  )


def _sc_gather(table, idx):
  V, D = table.shape
  B = idx.shape[0]
  info = plsc.get_sparse_core_info()
  NC, NS = info.num_cores, info.num_subcores
  NW = NC * NS
  assert B % NW == 0 and (B // NW) % 8 == 0
  b_per_w = B // NW

  mesh = plsc.VectorSubcoreMesh(core_axis_name="c", subcore_axis_name="s")

  @functools.partial(
      pl.kernel,
      mesh=mesh,
      out_type=jax.ShapeDtypeStruct((B, D), jnp.float32),
      scratch_types=[
          pltpu.VMEM((b_per_w,), jnp.int32),
          pltpu.VMEM((b_per_w, D), jnp.float32),
          pltpu.SemaphoreType.DMA,
      ],
      compiler_params=pltpu.CompilerParams(use_tc_tiling_on_sc=True),
  )
  def gather_kernel(table_hbm, idx_hbm, out_hbm, idx_v, rows_v, sem):
    wid = lax.axis_index("s") * NC + lax.axis_index("c")
    base = wid * b_per_w
    pltpu.sync_copy(idx_hbm.at[pl.ds(base, b_per_w)], idx_v)
    pltpu.async_copy(table_hbm.at[idx_v], rows_v, sem).wait()
    pltpu.sync_copy(rows_v, out_hbm.at[pl.ds(base, b_per_w)])

  return gather_kernel(table, idx)


_V_TILE = 2048
_NBUF = 4


def _make_proj_t(B, E1, V, D):
  ngrid = pl.cdiv(V, _V_TILE)
  tail = V - (ngrid - 1) * _V_TILE

  def body(wb_ref, emb_ref, out_hbm, *scratch):
    bufs = scratch[:_NBUF]
    sems = scratch[_NBUF:]
    i = pl.program_id(0)
    phase = jax.lax.rem(i, _NBUF)
    acc = jax.lax.dot_general(
        wb_ref[...],
        emb_ref[:, :E1],
        dimension_numbers=(((0,), (1,)), ((), ())),
        preferred_element_type=jnp.float32,
    )

    for k in range(_NBUF):
      @pl.when(phase == k)
      def _(k=k):
        @pl.when(i >= _NBUF)
        def _():
          pltpu.make_async_copy(
              bufs[k], out_hbm.at[pl.ds((i - _NBUF) * _V_TILE, _V_TILE), :],
              sems[k]).wait()
        bufs[k][...] = acc
        @pl.when(i < ngrid - 1)
        def _():
          pltpu.make_async_copy(
              bufs[k], out_hbm.at[pl.ds(i * _V_TILE, _V_TILE), :],
              sems[k]).start()
        @pl.when(i == ngrid - 1)
        def _():
          pltpu.make_async_copy(
              bufs[k].at[pl.ds(0, tail), :],
              out_hbm.at[pl.ds(i * _V_TILE, tail), :],
              sems[k]).start()
          pltpu.make_async_copy(
              bufs[k].at[pl.ds(0, tail), :],
              out_hbm.at[pl.ds(i * _V_TILE, tail), :],
              sems[k]).wait()

    @pl.when(i == ngrid - 1)
    def _():
      for k in range(_NBUF):
        @pl.when(phase != k)
        def _(k=k):
          pltpu.make_async_copy(
              bufs[k], out_hbm.at[pl.ds(0, _V_TILE), :], sems[k]).wait()

  return pl.pallas_call(
      body,
      grid=(ngrid,),
      in_specs=[
          pl.BlockSpec((E1, _V_TILE), lambda i: (0, i)),
          pl.BlockSpec((B, D), lambda i: (0, 0)),
      ],
      out_specs=pl.BlockSpec(memory_space=pl.ANY),
      out_shape=jax.ShapeDtypeStruct((V, B), jnp.float32),
      scratch_shapes=(
          [pltpu.VMEM((_V_TILE, B), jnp.float32) for _ in range(_NBUF)]
          + [pltpu.SemaphoreType.DMA for _ in range(_NBUF)]
      ),
      name="band_proj_t",
  )


def kernel(center_word, emb_table, W, b):
  idx = center_word.astype(jnp.int32)
  V, E = emb_table.shape
  B = idx.shape[0]
  table_t = jnp.swapaxes(emb_table, 0, 1)
  table_pad = _make_tpad(E, V)(table_t)
  emb_pad = _sc_gather(table_pad, idx)
  wt = jnp.swapaxes(W, 0, 1)
  wb = jnp.concatenate([wt, b[None, :]], axis=0)
  out_t = _make_proj_t(B, E + 1, V, _LANES)(wb, emb_pad)
  return jnp.swapaxes(out_t, 0, 1)
